# Initial kernel scaffold; baseline (speedup 1.0000x reference)
import jax
import jax.numpy as jnp
from jax.experimental import pallas as pl


def kernel(x, edge_index, edge_attr, eW1, eb1, eW2, eb2, nW1, nb1, nW2, nb2):
    raise NotImplementedError("write your pallas kernel here")


# speedup vs baseline: 3.7254x; 3.7254x over previous
import functools

import jax
import jax.numpy as jnp
from jax import lax
from jax.experimental import pallas as pl
from jax.experimental.pallas import tpu as pltpu
from jax.experimental.pallas import tpu_sc as plsc

N_NODES = 10000
N_EDGES = 320000
D = 128
DE = 16

NC = 2
NS = 16
NW = NC * NS
EPW = N_EDGES // NW
CH = 80
NCHUNK = EPW // CH
NP = 10240
RPT = NP // NS
NSUB = 2
NVREG = EPW // 16

_SC_MESH = plsc.VectorSubcoreMesh(
    core_axis_name="c", subcore_axis_name="s", num_cores=NC, num_subcores=NS)


def _gather_body(x_hbm, row_hbm, col_hbm, gr_hbm, gc_hbm,
                 row_v, col_v, gr_v, gc_v, sem_r, sem_c):
    wid = lax.axis_index("s") * NC + lax.axis_index("c")
    pltpu.sync_copy(row_hbm.at[wid], row_v)
    pltpu.sync_copy(col_hbm.at[wid], col_v)

    def body(i, carry):
        base = wid * EPW + i * CH
        cpr = pltpu.async_copy(x_hbm.at[row_v.at[i]], gr_v, sem_r)
        cpc = pltpu.async_copy(x_hbm.at[col_v.at[i]], gc_v, sem_c)
        cpr.wait()
        cpc.wait()
        pltpu.sync_copy(gr_v, gr_hbm.at[pl.ds(base, CH)])
        pltpu.sync_copy(gc_v, gc_hbm.at[pl.ds(base, CH)])
        return carry

    lax.fori_loop(0, NCHUNK, body, 0)


@functools.partial(
    pl.kernel,
    out_type=[
        jax.ShapeDtypeStruct((N_EDGES, D), jnp.float32),
        jax.ShapeDtypeStruct((N_EDGES, D), jnp.float32),
    ],
    mesh=_SC_MESH,
    scratch_types=[
        pltpu.VMEM((NCHUNK, CH), jnp.int32),
        pltpu.VMEM((NCHUNK, CH), jnp.int32),
        pltpu.VMEM((CH, D), jnp.float32),
        pltpu.VMEM((CH, D), jnp.float32),
        pltpu.SemaphoreType.DMA,
        pltpu.SemaphoreType.DMA,
    ],
)
def _gather(x, row3, col3, gr, gc, *scratch):
    _gather_body(x, row3, col3, gr, gc, *scratch)


def _edge_body(gr_ref, gc_ref, ea_ref, w1a_ref, w1b_ref, w1c_ref, b1_ref,
               w2_ref, b2_ref, nw1a_ref, nw1b_ref, nb1_ref,
               nea_ref, hn_ref):
    gr = gr_ref[...]
    f32 = jnp.float32
    ab = (jnp.dot(gr, w1a_ref[...], preferred_element_type=f32)
          + jnp.dot(gc_ref[...], w1b_ref[...], preferred_element_type=f32)
          + jnp.dot(ea_ref[...], w1c_ref[...], preferred_element_type=f32)
          + b1_ref[...])
    he = jnp.maximum(ab, 0.0)
    nea = jnp.dot(he, w2_ref[...], preferred_element_type=f32) + b2_ref[...]
    nea_ref[...] = nea
    hn_ref[...] = jnp.maximum(
        jnp.dot(gr, nw1a_ref[...], preferred_element_type=f32)
        + jnp.dot(nea, nw1b_ref[...], preferred_element_type=f32)
        + nb1_ref[...], 0.0)


def _edge(gr, gc, ea, w1a, w1b, w1c, b1, w2, b2, nw1a, nw1b, nb1):
    blk = 4000

    def full(shape):
        return pl.BlockSpec(shape, lambda i: (0, 0))

    return pl.pallas_call(
        _edge_body,
        grid=(N_EDGES // blk,),
        in_specs=[
            pl.BlockSpec((blk, D), lambda i: (i, 0)),
            pl.BlockSpec((blk, D), lambda i: (i, 0)),
            pl.BlockSpec((blk, DE), lambda i: (i, 0)),
            full((D, DE)),
            full((D, DE)),
            full((DE, DE)),
            full((1, DE)),
            full((DE, DE)),
            full((1, DE)),
            full((D, D)),
            full((DE, D)),
            full((1, D)),
        ],
        out_specs=[
            pl.BlockSpec((blk, DE), lambda i: (i, 0)),
            pl.BlockSpec((blk, D), lambda i: (i, 0)),
        ],
        out_shape=[
            jax.ShapeDtypeStruct((N_EDGES, DE), jnp.float32),
            jax.ShapeDtypeStruct((N_EDGES, D), jnp.float32),
        ],
    )(gr, gc, ea, w1a, w1b, w1c, b1, w2, b2, nw1a, nw1b, nb1)


def _scatter_body(hn_hbm, col_hbm, zacc_hbm, zcnt_hbm,
                  part_hbm, cnt_hbm,
                  col_v, hn_v, cnt8_v, acc):
    c_id = lax.axis_index("c")
    s_id = lax.axis_index("s")
    wid = s_id * NC + c_id
    rbase = s_id * RPT
    pltpu.sync_copy(zacc_hbm.at[pl.ds(rbase, RPT)], acc.at[pl.ds(rbase, RPT)])
    pltpu.sync_copy(zcnt_hbm, cnt8_v)
    pltpu.sync_copy(col_hbm.at[wid], col_v)
    plsc.subcore_barrier()

    def body(i, carry):
        base = wid * EPW + i * CH
        pltpu.sync_copy(hn_hbm.at[pl.ds(base, CH)], hn_v)
        pltpu.sync_copy(hn_v, acc.at[col_v.at[i]], add=True)
        return carry

    lax.fori_loop(0, NCHUNK, body, 0)

    lane = lax.iota(jnp.int32, 16)
    ioff = lax.rem(lane, NSUB) * NP
    group = lane // NSUB
    masks = [group == g for g in range(16 // NSUB)]
    ones16 = jnp.full((16,), 1.0, jnp.float32)
    npair = CH // 16

    def cbody(k, carry):
        i = k // npair
        j = k - i * npair
        idx = col_v[i, pl.ds(j * 16, 16)] + ioff
        for m in masks:
            plsc.addupdate_scatter(cnt8_v, [idx], ones16, mask=m)
        return carry

    lax.fori_loop(0, NVREG, cbody, 0)
    plsc.subcore_barrier()
    pltpu.sync_copy(acc.at[pl.ds(rbase, RPT)],
                    part_hbm.at[c_id].at[pl.ds(rbase, RPT)])
    pltpu.sync_copy(cnt8_v, cnt_hbm.at[wid])


@functools.partial(
    pl.kernel,
    out_type=[
        jax.ShapeDtypeStruct((NC, NP, D), jnp.float32),
        jax.ShapeDtypeStruct((NW, NSUB * NP), jnp.float32),
    ],
    mesh=_SC_MESH,
    scratch_types=[
        pltpu.VMEM((NCHUNK, CH), jnp.int32),
        pltpu.VMEM((CH, D), jnp.float32),
        pltpu.VMEM((NSUB * NP,), jnp.float32),
        pltpu.VMEM_SHARED((NP, D), jnp.float32),
    ],
    compiler_params=pltpu.CompilerParams(needs_layout_passes=False),
)
def _scatter(hn, col3, zacc, zcnt, part, cnt, *scratch):
    _scatter_body(hn, col3, zacc, zcnt, part, cnt, *scratch)


def _post_body(p0_ref, p1_ref, c_ref, ones_ref, w_ref, nb2_ref, out_ref):
    sums = p0_ref[...] + p1_ref[...]
    cnt = jnp.dot(c_ref[...], ones_ref[...],
                  preferred_element_type=jnp.float32)
    denom = jnp.maximum(cnt, 1.0)
    out_ref[...] = jnp.maximum(
        (jnp.dot(sums, w_ref[...], preferred_element_type=jnp.float32)
         + nb2_ref[...] * cnt) / denom, 0.0)


def _post(p0, p1, cnt_t, ones, w, nb2):
    blk = 1280
    nsh = NW * NSUB
    return pl.pallas_call(
        _post_body,
        grid=(NP // blk,),
        in_specs=[
            pl.BlockSpec((blk, D), lambda i: (i, 0)),
            pl.BlockSpec((blk, D), lambda i: (i, 0)),
            pl.BlockSpec((blk, nsh), lambda i: (i, 0)),
            pl.BlockSpec((nsh, 1), lambda i: (0, 0)),
            pl.BlockSpec((D, D), lambda i: (0, 0)),
            pl.BlockSpec((1, D), lambda i: (0, 0)),
        ],
        out_specs=pl.BlockSpec((blk, D), lambda i: (i, 0)),
        out_shape=jax.ShapeDtypeStruct((NP, D), jnp.float32),
    )(p0, p1, cnt_t, ones, w, nb2)


def kernel(x, edge_index, edge_attr, eW1, eb1, eW2, eb2, nW1, nb1, nW2, nb2):
    row = edge_index[0].astype(jnp.int32)
    col = edge_index[1].astype(jnp.int32)
    row3 = row.reshape(NW, NCHUNK, CH)
    col3 = col.reshape(NW, NCHUNK, CH)
    gr, gc = _gather(x, row3, col3)

    nea, hn = _edge(gr, gc, edge_attr,
                    eW1[:D], eW1[D:2 * D], eW1[2 * D:], eb1.reshape(1, DE),
                    eW2, eb2.reshape(1, DE),
                    nW1[:D], nW1[D:], nb1.reshape(1, D))

    zacc = jnp.zeros((NP, D), jnp.float32)
    zcnt = jnp.zeros((NSUB * NP,), jnp.float32)
    part, cnt = _scatter(hn, col3, zacc, zcnt)
    cnt_t = cnt.reshape(NW * NSUB, NP).T
    ones = jnp.ones((NW * NSUB, 1), jnp.float32)
    out = _post(part[0], part[1], cnt_t, ones, nW2, nb2.reshape(1, D))
    return out[:N_NODES], nea


# speedup vs baseline: 3.8045x; 1.0212x over previous
import functools

import jax
import jax.numpy as jnp
from jax import lax
from jax.experimental import pallas as pl
from jax.experimental.pallas import tpu as pltpu
from jax.experimental.pallas import tpu_sc as plsc

N_NODES = 10000
N_EDGES = 320000
D = 128
DE = 16

NC = 2
NS = 16
NW = NC * NS
EPW = N_EDGES // NW
CH = 80
NCHUNK = EPW // CH
NP = 10240
RPT = NP // NS
NSUB = 2
NVREG = EPW // 16

_SC_MESH = plsc.VectorSubcoreMesh(
    core_axis_name="c", subcore_axis_name="s", num_cores=NC, num_subcores=NS)


def _gather_body(x_hbm, row_hbm, col_hbm, gr_hbm, gc_hbm,
                 row_v, col_v, gr_v, gc_v, sem_r, sem_c):
    wid = lax.axis_index("s") * NC + lax.axis_index("c")
    pltpu.sync_copy(row_hbm.at[wid], row_v)
    pltpu.sync_copy(col_hbm.at[wid], col_v)

    def body(i, carry):
        base = wid * EPW + i * CH
        cpr = pltpu.async_copy(x_hbm.at[row_v.at[i]], gr_v, sem_r)
        cpc = pltpu.async_copy(x_hbm.at[col_v.at[i]], gc_v, sem_c)
        cpr.wait()
        cpc.wait()
        pltpu.sync_copy(gr_v, gr_hbm.at[pl.ds(base, CH)])
        pltpu.sync_copy(gc_v, gc_hbm.at[pl.ds(base, CH)])
        return carry

    lax.fori_loop(0, NCHUNK, body, 0)


@functools.partial(
    pl.kernel,
    out_type=[
        jax.ShapeDtypeStruct((N_EDGES, D), jnp.float32),
        jax.ShapeDtypeStruct((N_EDGES, D), jnp.float32),
    ],
    mesh=_SC_MESH,
    scratch_types=[
        pltpu.VMEM((NCHUNK, CH), jnp.int32),
        pltpu.VMEM((NCHUNK, CH), jnp.int32),
        pltpu.VMEM((CH, D), jnp.float32),
        pltpu.VMEM((CH, D), jnp.float32),
        pltpu.SemaphoreType.DMA,
        pltpu.SemaphoreType.DMA,
    ],
)
def _gather(x, row3, col3, gr, gc, *scratch):
    _gather_body(x, row3, col3, gr, gc, *scratch)


def _edge_body(gr_ref, gc_ref, ea_ref, w1a_ref, w1b_ref, w1c_ref, b1_ref,
               w2_ref, b2_ref, nw1a_ref, nw1b_ref, nb1_ref,
               nea_ref, hn_ref):
    f32 = jnp.float32
    bf16 = jnp.bfloat16
    gr = gr_ref[...].astype(bf16)
    gc = gc_ref[...].astype(bf16)
    ab = (jnp.dot(gr, w1a_ref[...], preferred_element_type=f32)
          + jnp.dot(gc, w1b_ref[...], preferred_element_type=f32)
          + jnp.dot(ea_ref[...], w1c_ref[...], preferred_element_type=f32)
          + b1_ref[...])
    he = jnp.maximum(ab, 0.0).astype(bf16)
    nea = jnp.dot(he, w2_ref[...], preferred_element_type=f32) + b2_ref[...]
    nea_ref[...] = nea
    hn_ref[...] = jnp.maximum(
        jnp.dot(gr, nw1a_ref[...], preferred_element_type=f32)
        + jnp.dot(nea.astype(bf16), nw1b_ref[...], preferred_element_type=f32)
        + nb1_ref[...], 0.0)


def _edge(gr, gc, ea, w1a, w1b, w1c, b1, w2, b2, nw1a, nw1b, nb1):
    blk = 4000

    def full(shape):
        return pl.BlockSpec(shape, lambda i: (0, 0))

    return pl.pallas_call(
        _edge_body,
        grid=(N_EDGES // blk,),
        in_specs=[
            pl.BlockSpec((blk, D), lambda i: (i, 0)),
            pl.BlockSpec((blk, D), lambda i: (i, 0)),
            pl.BlockSpec((blk, DE), lambda i: (i, 0)),
            full((D, DE)),
            full((D, DE)),
            full((DE, DE)),
            full((1, DE)),
            full((DE, DE)),
            full((1, DE)),
            full((D, D)),
            full((DE, D)),
            full((1, D)),
        ],
        out_specs=[
            pl.BlockSpec((blk, DE), lambda i: (i, 0)),
            pl.BlockSpec((blk, D), lambda i: (i, 0)),
        ],
        out_shape=[
            jax.ShapeDtypeStruct((N_EDGES, DE), jnp.float32),
            jax.ShapeDtypeStruct((N_EDGES, D), jnp.float32),
        ],
    )(gr, gc, ea, w1a, w1b, w1c, b1, w2, b2, nw1a, nw1b, nb1)


def _scatter_body(hn_hbm, col_hbm, zacc_hbm, zcnt_hbm,
                  part_hbm, cnt_hbm,
                  col_v, hn_v, cnt8_v, acc):
    c_id = lax.axis_index("c")
    s_id = lax.axis_index("s")
    wid = s_id * NC + c_id
    rbase = s_id * RPT
    pltpu.sync_copy(zacc_hbm.at[pl.ds(rbase, RPT)], acc.at[pl.ds(rbase, RPT)])
    pltpu.sync_copy(zcnt_hbm, cnt8_v)
    pltpu.sync_copy(col_hbm.at[wid], col_v)
    plsc.subcore_barrier()

    def body(i, carry):
        base = wid * EPW + i * CH
        pltpu.sync_copy(hn_hbm.at[pl.ds(base, CH)], hn_v)
        pltpu.sync_copy(hn_v, acc.at[col_v.at[i]], add=True)
        return carry

    lax.fori_loop(0, NCHUNK, body, 0)

    lane = lax.iota(jnp.int32, 16)
    ioff = lax.rem(lane, NSUB) * NP
    group = lane // NSUB
    masks = [group == g for g in range(16 // NSUB)]
    ones16 = jnp.full((16,), 1.0, jnp.float32)
    npair = CH // 16

    def cbody(k, carry):
        i = k // npair
        j = k - i * npair
        idx = col_v[i, pl.ds(j * 16, 16)] + ioff
        for m in masks:
            plsc.addupdate_scatter(cnt8_v, [idx], ones16, mask=m)
        return carry

    lax.fori_loop(0, NVREG, cbody, 0)
    plsc.subcore_barrier()
    pltpu.sync_copy(acc.at[pl.ds(rbase, RPT)],
                    part_hbm.at[c_id].at[pl.ds(rbase, RPT)])
    pltpu.sync_copy(cnt8_v, cnt_hbm.at[wid])


@functools.partial(
    pl.kernel,
    out_type=[
        jax.ShapeDtypeStruct((NC, NP, D), jnp.float32),
        jax.ShapeDtypeStruct((NW, NSUB * NP), jnp.float32),
    ],
    mesh=_SC_MESH,
    scratch_types=[
        pltpu.VMEM((NCHUNK, CH), jnp.int32),
        pltpu.VMEM((CH, D), jnp.float32),
        pltpu.VMEM((NSUB * NP,), jnp.float32),
        pltpu.VMEM_SHARED((NP, D), jnp.float32),
    ],
    compiler_params=pltpu.CompilerParams(needs_layout_passes=False),
)
def _scatter(hn, col3, zacc, zcnt, part, cnt, *scratch):
    _scatter_body(hn, col3, zacc, zcnt, part, cnt, *scratch)


def _post_body(p0_ref, p1_ref, c_ref, ones_ref, w_ref, nb2_ref, out_ref):
    sums = p0_ref[...] + p1_ref[...]
    cnt = jnp.dot(c_ref[...], ones_ref[...],
                  preferred_element_type=jnp.float32)
    denom = jnp.maximum(cnt, 1.0)
    out_ref[...] = jnp.maximum(
        (jnp.dot(sums, w_ref[...], preferred_element_type=jnp.float32)
         + nb2_ref[...] * cnt) / denom, 0.0)


def _post(p0, p1, cnt_t, ones, w, nb2):
    blk = 1280
    nsh = NW * NSUB
    return pl.pallas_call(
        _post_body,
        grid=(NP // blk,),
        in_specs=[
            pl.BlockSpec((blk, D), lambda i: (i, 0)),
            pl.BlockSpec((blk, D), lambda i: (i, 0)),
            pl.BlockSpec((blk, nsh), lambda i: (i, 0)),
            pl.BlockSpec((nsh, 1), lambda i: (0, 0)),
            pl.BlockSpec((D, D), lambda i: (0, 0)),
            pl.BlockSpec((1, D), lambda i: (0, 0)),
        ],
        out_specs=pl.BlockSpec((blk, D), lambda i: (i, 0)),
        out_shape=jax.ShapeDtypeStruct((NP, D), jnp.float32),
    )(p0, p1, cnt_t, ones, w, nb2)


def kernel(x, edge_index, edge_attr, eW1, eb1, eW2, eb2, nW1, nb1, nW2, nb2):
    row = edge_index[0].astype(jnp.int32)
    col = edge_index[1].astype(jnp.int32)
    row3 = row.reshape(NW, NCHUNK, CH)
    col3 = col.reshape(NW, NCHUNK, CH)
    bf16 = jnp.bfloat16
    gr, gc = _gather(x, row3, col3)

    nea, hn = _edge(gr, gc, edge_attr.astype(bf16),
                    eW1[:D].astype(bf16), eW1[D:2 * D].astype(bf16),
                    eW1[2 * D:].astype(bf16), eb1.reshape(1, DE),
                    eW2.astype(bf16), eb2.reshape(1, DE),
                    nW1[:D].astype(bf16), nW1[D:].astype(bf16),
                    nb1.reshape(1, D))

    zacc = jnp.zeros((NP, D), jnp.float32)
    zcnt = jnp.zeros((NSUB * NP,), jnp.float32)
    part, cnt = _scatter(hn, col3, zacc, zcnt)
    cnt_t = cnt.reshape(NW * NSUB, NP).T
    ones = jnp.ones((NW * NSUB, 1), jnp.float32)
    out = _post(part[0], part[1], cnt_t, ones, nW2, nb2.reshape(1, D))
    return out[:N_NODES], nea
